# Initial kernel scaffold; baseline (speedup 1.0000x reference)
import jax
import jax.numpy as jnp
from jax.experimental import pallas as pl


def kernel(weight, allfeature, keyfeature, refinepoint, keypoint, topidx, k, conv_w, conv_b, bn_g, bn_b):
    raise NotImplementedError("write your pallas kernel here")


# speedup vs baseline: 22.9341x; 22.9341x over previous
import functools

import jax
import jax.numpy as jnp
from jax import lax
from jax.experimental import pallas as pl
from jax.experimental.pallas import tpu as pltpu
from jax.experimental.pallas import tpu_sc as plsc

B, N, KP, C, KNN = 8, 2048, 512, 128, 20

NW = 32
QPW = (B * KP) // NW
CH = 4
NCH = QPW // CH


def _knn_body(w_ref, af_ref, rp_ref, rpt_ref, ti_ref, allf_ref, gidx_ref):
    b = pl.program_id(0)
    allf_ref[0] = af_ref[0] * w_ref[0]
    x = rp_ref[0]
    xt = rpt_ref[0]
    ti = ti_ref[0]
    iota_n = lax.broadcasted_iota(jnp.int32, (KP, N), 1)
    onehot = (iota_n == ti).astype(jnp.float32)
    q = jnp.dot(onehot, x, preferred_element_type=jnp.float32,
                precision=lax.Precision.HIGHEST)
    xxq = jnp.sum(q * q, axis=1, keepdims=True)
    xt0, xt1, xt2 = xt[0:1], xt[1:2], xt[2:3]
    xx_row = xt0 * xt0 + xt1 * xt1 + xt2 * xt2
    def _r(v):
        return v.astype(jnp.bfloat16).astype(jnp.float32)
    inner = (_r(q[:, 0:1]) * _r(xt0) + _r(q[:, 1:2]) * _r(xt1)
             + _r(q[:, 2:3]) * _r(xt2))
    d = (2.0 * inner - xxq) - xx_row
    neg_inf = jnp.float32(-jnp.inf)
    for j in range(KNN):
        m = jnp.max(d, axis=1, keepdims=True)
        cand = jnp.where(d == m, iota_n, N)
        sel = jnp.min(cand, axis=1, keepdims=True)
        gidx_ref[0, :, j:j + 1] = sel + b * N
        d = jnp.where(iota_n == sel, neg_inf, d)


def _knn_call(w3, allfeature, rp, rpt, ti3):
    return pl.pallas_call(
        _knn_body,
        grid=(B,),
        in_specs=[
            pl.BlockSpec((1, N, 1), lambda b: (b, 0, 0)),
            pl.BlockSpec((1, N, C), lambda b: (b, 0, 0)),
            pl.BlockSpec((1, N, 3), lambda b: (b, 0, 0)),
            pl.BlockSpec((1, 3, N), lambda b: (b, 0, 0)),
            pl.BlockSpec((1, KP, 1), lambda b: (b, 0, 0)),
        ],
        out_specs=[
            pl.BlockSpec((1, N, C), lambda b: (b, 0, 0)),
            pl.BlockSpec((1, KP, KNN), lambda b: (b, 0, 0)),
        ],
        out_shape=[
            jax.ShapeDtypeStruct((B, N, C), jnp.float32),
            jax.ShapeDtypeStruct((B, KP, KNN), jnp.int32),
        ],
    )(w3, allfeature, rp, rpt, ti3)


def _sc_fuse_call(allf_flat, gidx_flat, keyf_flat):
    mesh = plsc.VectorSubcoreMesh(core_axis_name="c", subcore_axis_name="s")

    @functools.partial(
        pl.kernel,
        mesh=mesh,
        out_type=jax.ShapeDtypeStruct((B * KP, C), jnp.float32),
        scratch_types=[
            pltpu.VMEM((CH * KNN,), jnp.int32),
            pltpu.VMEM((CH * KNN, C), jnp.float32),
            pltpu.VMEM((CH, C), jnp.float32),
            pltpu.VMEM((CH, C), jnp.float32),
            pltpu.SemaphoreType.DMA,
        ],
    )
    def _fuse(allf_hbm, gidx_hbm, keyf_hbm, out_hbm,
              idx_v, rows_v, keyf_v, out_v, sem):
        wid = lax.axis_index("s") * 2 + lax.axis_index("c")

        def chunk(ic, carry):
            qbase = wid * QPW + ic * CH
            pltpu.sync_copy(gidx_hbm.at[pl.ds(qbase * KNN, CH * KNN)], idx_v)
            pltpu.async_copy(allf_hbm.at[idx_v], rows_v, sem).wait()
            pltpu.sync_copy(keyf_hbm.at[pl.ds(qbase, CH)], keyf_v)

            def per_q(qi, c2):
                for cc in range(C // 16):
                    sl = pl.ds(cc * 16, 16)
                    acc = rows_v[qi * KNN, sl]
                    for j in range(1, KNN):
                        acc = acc + rows_v[qi * KNN + j, sl]
                    out_v[qi, sl] = acc / jnp.float32(KNN) + keyf_v[qi, sl]
                return c2

            lax.fori_loop(0, CH, per_q, 0)
            pltpu.sync_copy(out_v, out_hbm.at[pl.ds(qbase, CH)])
            return carry

        lax.fori_loop(0, NCH, chunk, 0)

    return _fuse(allf_flat, gidx_flat, keyf_flat)


def _head_body(f_ref, w_ref, cb_ref, g_ref, bb_ref, o_ref):
    f = f_ref[...]
    y = lax.dot_general(f, w_ref[...], (((1,), (1,)), ((), ())),
                        preferred_element_type=jnp.float32) + cb_ref[...]
    mean = jnp.mean(y, axis=0, keepdims=True)
    var = jnp.mean((y - mean) ** 2, axis=0, keepdims=True)
    yn = (y - mean) / jnp.sqrt(var + 1e-5)
    yn = yn * g_ref[...] + bb_ref[...]
    o_ref[...] = jnp.where(yn >= 0, yn, 0.2 * yn)


def _head_call(fused, conv_w, conv_b, bn_g, bn_b):
    return pl.pallas_call(
        _head_body,
        out_shape=jax.ShapeDtypeStruct((B * KP, C), jnp.float32),
    )(fused, conv_w, conv_b.reshape(1, C), bn_g.reshape(1, C),
      bn_b.reshape(1, C))


def kernel(weight, allfeature, keyfeature, refinepoint, keypoint, topidx, k,
           conv_w, conv_b, bn_g, bn_b):
    w3 = weight.reshape(B, N, 1)
    ti3 = topidx.astype(jnp.int32).reshape(B, KP, 1)
    rpt = refinepoint.transpose(0, 2, 1)
    allf, gidx = _knn_call(w3, allfeature, refinepoint, rpt, ti3)
    fused = _sc_fuse_call(allf.reshape(B * N, C),
                          gidx.reshape(B * KP * KNN),
                          keyfeature.reshape(B * KP, C))
    out = _head_call(fused, conv_w, conv_b, bn_g, bn_b)
    return out.reshape(B, KP, C).transpose(0, 2, 1)


# speedup vs baseline: 27.6894x; 1.2073x over previous
import functools

import jax
import jax.numpy as jnp
from jax import lax
from jax.experimental import pallas as pl
from jax.experimental.pallas import tpu as pltpu
from jax.experimental.pallas import tpu_sc as plsc

B, N, KP, C, KNN = 8, 2048, 512, 128, 20

NW = 32
QPW = (B * KP) // NW
CH = 4
NCH = QPW // CH


def _knn_body(w_ref, af_ref, rp_ref, rpt_ref, ti_ref, allf_ref, gidx_ref):
    b = pl.program_id(0)
    allf_ref[0] = af_ref[0] * w_ref[0]
    x = rp_ref[0]
    xt = rpt_ref[0]
    ti = ti_ref[0]
    iota_n = lax.broadcasted_iota(jnp.int32, (KP, N), 1)
    onehot = (iota_n == ti).astype(jnp.float32)
    q = jnp.dot(onehot, x, preferred_element_type=jnp.float32,
                precision=lax.Precision.HIGHEST)
    xxq = jnp.sum(q * q, axis=1, keepdims=True)
    xt0, xt1, xt2 = xt[0:1], xt[1:2], xt[2:3]
    xx_row = xt0 * xt0 + xt1 * xt1 + xt2 * xt2
    def _r(v):
        return v.astype(jnp.bfloat16).astype(jnp.float32)
    inner = (_r(q[:, 0:1]) * _r(xt0) + _r(q[:, 1:2]) * _r(xt1)
             + _r(q[:, 2:3]) * _r(xt2))
    d = (2.0 * inner - xxq) - xx_row
    neg_inf = jnp.float32(-jnp.inf)
    for j in range(KNN):
        m = jnp.max(d, axis=1, keepdims=True)
        cand = jnp.where(d == m, iota_n, N)
        sel = jnp.min(cand, axis=1, keepdims=True)
        gidx_ref[0, :, j:j + 1] = sel + b * N
        d = jnp.where(iota_n == sel, neg_inf, d)


def _knn_call(w3, allfeature, rp, rpt, ti3):
    return pl.pallas_call(
        _knn_body,
        grid=(B,),
        in_specs=[
            pl.BlockSpec((1, N, 1), lambda b: (b, 0, 0)),
            pl.BlockSpec((1, N, C), lambda b: (b, 0, 0)),
            pl.BlockSpec((1, N, 3), lambda b: (b, 0, 0)),
            pl.BlockSpec((1, 3, N), lambda b: (b, 0, 0)),
            pl.BlockSpec((1, KP, 1), lambda b: (b, 0, 0)),
        ],
        out_specs=[
            pl.BlockSpec((1, N, C), lambda b: (b, 0, 0)),
            pl.BlockSpec((1, KP, KNN), lambda b: (b, 0, 0)),
        ],
        out_shape=[
            jax.ShapeDtypeStruct((B, N, C), jnp.float32),
            jax.ShapeDtypeStruct((B, KP, KNN), jnp.int32),
        ],
    )(w3, allfeature, rp, rpt, ti3)


def _sc_fuse_call(allf_flat, gidx_flat, keyf_flat):
    mesh = plsc.VectorSubcoreMesh(core_axis_name="c", subcore_axis_name="s")

    @functools.partial(
        pl.kernel,
        mesh=mesh,
        out_type=jax.ShapeDtypeStruct((B * KP, C), jnp.float32),
        scratch_types=[
            pltpu.VMEM((QPW * KNN,), jnp.int32),
            pltpu.VMEM((2, CH * KNN, C), jnp.float32),
            pltpu.VMEM((QPW, C), jnp.float32),
            pltpu.VMEM((QPW, C), jnp.float32),
            pltpu.SemaphoreType.DMA,
            pltpu.SemaphoreType.DMA,
            pltpu.SemaphoreType.DMA,
        ],
    )
    def _fuse(allf_hbm, gidx_hbm, keyf_hbm, out_hbm,
              idx_v, rows_v, keyf_v, out_v, sem0, sem1, semk):
        wid = lax.axis_index("s") * 2 + lax.axis_index("c")
        qbase = wid * QPW
        sems = (sem0, sem1)
        pltpu.sync_copy(gidx_hbm.at[pl.ds(qbase * KNN, QPW * KNN)], idx_v)
        kcp = pltpu.async_copy(keyf_hbm.at[pl.ds(qbase, QPW)], keyf_v, semk)

        def _fire(c, buf, sem):
            pltpu.async_copy(
                allf_hbm.at[idx_v.at[pl.ds(c * (CH * KNN), CH * KNN)]],
                rows_v.at[buf], sem)

        def _drain(buf, sem):
            pltpu.make_async_copy(
                allf_hbm.at[idx_v.at[pl.ds(0, CH * KNN)]],
                rows_v.at[buf], sem).wait()

        _fire(0, 0, sem0)
        _fire(1, 1, sem1)
        kcp.wait()

        def pair(ip, carry):
            for bpar in range(2):
                c = 2 * ip + bpar
                sem = sems[bpar]
                _drain(bpar, sem)

                def per_q(qi, c2):
                    rows = rows_v.at[bpar]
                    for cc in range(C // 16):
                        sl = pl.ds(cc * 16, 16)
                        acc = rows[qi * KNN, sl]
                        for j in range(1, KNN):
                            acc = acc + rows[qi * KNN + j, sl]
                        out_v[c * CH + qi, sl] = (acc / jnp.float32(KNN)
                                                  + keyf_v[c * CH + qi, sl])
                    return c2

                lax.fori_loop(0, CH, per_q, 0)

                @pl.when(c + 2 < NCH)
                def _():
                    _fire(c + 2, bpar, sem)
            return carry

        lax.fori_loop(0, NCH // 2, pair, 0)
        pltpu.sync_copy(out_v, out_hbm.at[pl.ds(qbase, QPW)])

    return _fuse(allf_flat, gidx_flat, keyf_flat)


def _head_body(f_ref, w_ref, cb_ref, g_ref, bb_ref, o_ref):
    f = f_ref[...]
    y = lax.dot_general(f, w_ref[...], (((1,), (1,)), ((), ())),
                        preferred_element_type=jnp.float32) + cb_ref[...]
    mean = jnp.mean(y, axis=0, keepdims=True)
    var = jnp.mean((y - mean) ** 2, axis=0, keepdims=True)
    yn = (y - mean) / jnp.sqrt(var + 1e-5)
    yn = yn * g_ref[...] + bb_ref[...]
    o_ref[...] = jnp.where(yn >= 0, yn, 0.2 * yn)


def _head_call(fused, conv_w, conv_b, bn_g, bn_b):
    return pl.pallas_call(
        _head_body,
        out_shape=jax.ShapeDtypeStruct((B * KP, C), jnp.float32),
    )(fused, conv_w, conv_b.reshape(1, C), bn_g.reshape(1, C),
      bn_b.reshape(1, C))


def kernel(weight, allfeature, keyfeature, refinepoint, keypoint, topidx, k,
           conv_w, conv_b, bn_g, bn_b):
    w3 = weight.reshape(B, N, 1)
    ti3 = topidx.astype(jnp.int32).reshape(B, KP, 1)
    rpt = refinepoint.transpose(0, 2, 1)
    allf, gidx = _knn_call(w3, allfeature, refinepoint, rpt, ti3)
    fused = _sc_fuse_call(allf.reshape(B * N, C),
                          gidx.reshape(B * KP * KNN),
                          keyfeature.reshape(B * KP, C))
    out = _head_call(fused, conv_w, conv_b, bn_g, bn_b)
    return out.reshape(B, KP, C).transpose(0, 2, 1)
